# Initial kernel scaffold; baseline (speedup 1.0000x reference)
import jax
import jax.numpy as jnp
from jax.experimental import pallas as pl


def kernel(indices, W):
    raise NotImplementedError("write your pallas kernel here")


# speedup vs baseline: 1.0228x; 1.0228x over previous
import functools
import math

import jax
import jax.numpy as jnp
from jax import lax
from jax.experimental import pallas as pl
from jax.experimental.pallas import tpu as pltpu
from jax.experimental.pallas import tpu_sc as plsc

D_MODEL = 128
NUM_CORES = 2
NUM_SUBCORES = 16
NW = NUM_CORES * NUM_SUBCORES
ROWS = 4096 * 200
B_PER_W = ROWS // NW
CHUNK = 128
NCHUNK = B_PER_W // CHUNK
SCALE = math.sqrt(float(D_MODEL))

_mesh = plsc.VectorSubcoreMesh(core_axis_name="c", subcore_axis_name="s")


@functools.partial(
    pl.kernel,
    out_type=jax.ShapeDtypeStruct((ROWS, D_MODEL), jnp.float32),
    mesh=_mesh,
    scratch_types=[
        pltpu.VMEM((NCHUNK, CHUNK), jnp.int32),
        pltpu.VMEM((CHUNK, D_MODEL), jnp.float32),
        pltpu.SemaphoreType.DMA,
    ],
)
def _embed(table_hbm, idx_hbm, out_hbm, idx_v, buf, sem):
    wid = lax.axis_index("s") * NUM_CORES + lax.axis_index("c")
    base = wid * B_PER_W
    pltpu.sync_copy(idx_hbm.at[wid], idx_v)

    def chunk_body(j, carry):
        pltpu.async_copy(table_hbm.at[idx_v.at[j]], buf, sem).wait()

        def row_body(r, c2):
            for s in range(D_MODEL // 16):
                sl = pl.ds(s * 16, 16)
                buf[r, sl] = buf[r, sl] * SCALE
            return c2

        lax.fori_loop(0, CHUNK, row_body, 0)
        pltpu.sync_copy(buf, out_hbm.at[pl.ds(base + j * CHUNK, CHUNK)])
        return carry

    lax.fori_loop(0, NCHUNK, chunk_body, 0)


def kernel(indices, W):
    idx = indices.astype(jnp.int32).reshape(NW, NCHUNK, CHUNK)
    out = _embed(W, idx)
    return out.reshape(indices.shape + (D_MODEL,))


# speedup vs baseline: 1.6012x; 1.5655x over previous
import functools
import math

import jax
import jax.numpy as jnp
from jax import lax
from jax.experimental import pallas as pl
from jax.experimental.pallas import tpu as pltpu
from jax.experimental.pallas import tpu_sc as plsc

D_MODEL = 128
NUM_CORES = 2
NUM_SUBCORES = 16
NW = NUM_CORES * NUM_SUBCORES
ROWS = 4096 * 200
B_PER_W = ROWS // NW
CHUNK = 128
NCHUNK = B_PER_W // CHUNK
SCALE = math.sqrt(float(D_MODEL))

_mesh = plsc.VectorSubcoreMesh(core_axis_name="c", subcore_axis_name="s")


@functools.partial(
    pl.kernel,
    out_type=jax.ShapeDtypeStruct((ROWS, D_MODEL), jnp.float32),
    mesh=_mesh,
    scratch_types=[
        pltpu.VMEM((NCHUNK, CHUNK), jnp.int32),
        pltpu.VMEM((CHUNK, D_MODEL), jnp.float32),
        pltpu.VMEM((CHUNK, D_MODEL), jnp.float32),
        pltpu.SemaphoreType.DMA,
        pltpu.SemaphoreType.DMA,
    ],
)
def _embed(table_hbm, idx_hbm, out_hbm, idx_v, buf_a, buf_b, gsem, osem):
    wid = lax.axis_index("s") * NUM_CORES + lax.axis_index("c")
    base = wid * B_PER_W
    pltpu.sync_copy(idx_hbm.at[wid], idx_v)

    def g_start(j, buf):
        pltpu.make_async_copy(table_hbm.at[idx_v.at[j]], buf, gsem).start()

    def g_wait(buf):
        pltpu.make_async_copy(table_hbm.at[idx_v.at[0]], buf, gsem).wait()

    def o_start(j, buf):
        pltpu.make_async_copy(
            buf, out_hbm.at[pl.ds(base + j * CHUNK, CHUNK)], osem
        ).start()

    def o_wait(buf):
        pltpu.make_async_copy(buf, out_hbm.at[pl.ds(base, CHUNK)], osem).wait()

    def scale(buf):
        def row_body(r, c2):
            for s in range(D_MODEL // 16):
                sl = pl.ds(s * 16, 16)
                buf[r, sl] = buf[r, sl] * SCALE
            return c2

        lax.fori_loop(0, CHUNK, row_body, 0)

    g_start(0, buf_a)
    g_wait(buf_a)
    g_start(1, buf_b)
    scale(buf_a)
    o_start(0, buf_a)

    def pair_body(p, carry):
        j = 2 * p + 1
        g_wait(buf_b)
        o_wait(buf_a)
        g_start(j + 1, buf_a)
        scale(buf_b)
        o_start(j, buf_b)
        g_wait(buf_a)
        o_wait(buf_b)
        g_start(j + 2, buf_b)
        scale(buf_a)
        o_start(j + 1, buf_a)
        return carry

    lax.fori_loop(0, (NCHUNK - 2) // 2, pair_body, 0)

    g_wait(buf_b)
    o_wait(buf_a)
    scale(buf_b)
    o_start(NCHUNK - 1, buf_b)
    o_wait(buf_b)


def kernel(indices, W):
    idx = indices.astype(jnp.int32).reshape(NW, NCHUNK, CHUNK)
    out = _embed(W, idx)
    return out.reshape(indices.shape + (D_MODEL,))


# speedup vs baseline: 1.8422x; 1.1505x over previous
import functools
import math

import jax
import jax.numpy as jnp
from jax import lax
from jax.experimental import pallas as pl
from jax.experimental.pallas import tpu as pltpu
from jax.experimental.pallas import tpu_sc as plsc

D_MODEL = 128
NUM_CORES = 2
NUM_SUBCORES = 16
NW = NUM_CORES * NUM_SUBCORES
ROWS = 4096 * 200
B_PER_W = ROWS // NW
CHUNK = 128
NCHUNK = B_PER_W // CHUNK
SCALE = math.sqrt(float(D_MODEL))

_mesh = plsc.VectorSubcoreMesh(core_axis_name="c", subcore_axis_name="s")


NB = 4


@functools.partial(
    pl.kernel,
    out_type=jax.ShapeDtypeStruct((ROWS, D_MODEL), jnp.float32),
    mesh=_mesh,
    scratch_types=[
        pltpu.VMEM((NCHUNK, CHUNK), jnp.int32),
        pltpu.VMEM((CHUNK, D_MODEL), jnp.float32),
        pltpu.VMEM((CHUNK, D_MODEL), jnp.float32),
        pltpu.VMEM((CHUNK, D_MODEL), jnp.float32),
        pltpu.VMEM((CHUNK, D_MODEL), jnp.float32),
        pltpu.SemaphoreType.DMA,
        pltpu.SemaphoreType.DMA,
    ],
)
def _embed(table_hbm, idx_hbm, out_hbm, idx_v, b0, b1, b2, b3, gsem, osem):
    bufs = (b0, b1, b2, b3)
    wid = lax.axis_index("s") * NUM_CORES + lax.axis_index("c")
    base = wid * B_PER_W
    pltpu.sync_copy(idx_hbm.at[wid], idx_v)

    def g_start(j, buf):
        pltpu.make_async_copy(table_hbm.at[idx_v.at[j]], buf, gsem).start()

    def g_wait(buf):
        pltpu.make_async_copy(table_hbm.at[idx_v.at[0]], buf, gsem).wait()

    def o_start(j, buf):
        pltpu.make_async_copy(
            buf, out_hbm.at[pl.ds(base + j * CHUNK, CHUNK)], osem
        ).start()

    def o_wait():
        pltpu.make_async_copy(b0, out_hbm.at[pl.ds(base, CHUNK)], osem).wait()

    def scale(buf):
        @plsc.parallel_loop(0, CHUNK, step=1, unroll=4)
        def _row(r):
            for s in range(D_MODEL // 16):
                sl = pl.ds(s * 16, 16)
                buf[r, sl] = buf[r, sl] * SCALE

    for j in range(NB - 1):
        g_start(j, bufs[j])
    for j in range(NB):
        buf = bufs[j]
        g_wait(buf)
        scale(buf)
        o_start(j, buf)
        if j >= 1:
            o_wait()
        g_start(j + 3, bufs[(j + 3) % NB])

    def group_body(gi, carry):
        j0 = NB * gi
        for b in range(NB):
            j = j0 + b
            buf = bufs[b]
            g_wait(buf)
            scale(buf)
            o_start(j, buf)
            o_wait()
            g_start(j + 3, bufs[(b + 3) % NB])
        return carry

    lax.fori_loop(1, NCHUNK // NB - 1, group_body, 0)

    for j in range(NCHUNK - NB, NCHUNK):
        buf = bufs[j % NB]
        g_wait(buf)
        scale(buf)
        o_start(j, buf)
        o_wait()
        if j == NCHUNK - NB:
            g_start(NCHUNK - 1, bufs[(NCHUNK - 1) % NB])
    o_wait()


def kernel(indices, W):
    idx = indices.astype(jnp.int32).reshape(NW, NCHUNK, CHUNK)
    out = _embed(W, idx)
    return out.reshape(indices.shape + (D_MODEL,))


# speedup vs baseline: 1.8460x; 1.0021x over previous
import functools
import math

import jax
import jax.numpy as jnp
from jax import lax
from jax.experimental import pallas as pl
from jax.experimental.pallas import tpu as pltpu
from jax.experimental.pallas import tpu_sc as plsc

D_MODEL = 128
NUM_CORES = 2
NUM_SUBCORES = 16
NW = NUM_CORES * NUM_SUBCORES
ROWS = 4096 * 200
B_PER_W = ROWS // NW
CHUNK = 128
NCHUNK = B_PER_W // CHUNK
SCALE = math.sqrt(float(D_MODEL))

_mesh = plsc.VectorSubcoreMesh(core_axis_name="c", subcore_axis_name="s")


NB = 3
OCHUNK = 2 * CHUNK
NOUTER = B_PER_W // OCHUNK


@functools.partial(
    pl.kernel,
    out_type=jax.ShapeDtypeStruct((ROWS, D_MODEL), jnp.float32),
    mesh=_mesh,
    scratch_types=[
        pltpu.VMEM((NCHUNK, CHUNK), jnp.int32),
        pltpu.VMEM((OCHUNK, D_MODEL), jnp.float32),
        pltpu.VMEM((OCHUNK, D_MODEL), jnp.float32),
        pltpu.VMEM((OCHUNK, D_MODEL), jnp.float32),
        pltpu.SemaphoreType.DMA,
        pltpu.SemaphoreType.DMA,
    ],
)
def _embed(table_hbm, idx_hbm, out_hbm, idx_v, b0, b1, b2, gsem, osem):
    bufs = (b0, b1, b2)
    wid = lax.axis_index("s") * NUM_CORES + lax.axis_index("c")
    base = wid * B_PER_W
    pltpu.sync_copy(idx_hbm.at[wid], idx_v)

    def g_start(j, buf):
        pltpu.make_async_copy(
            table_hbm.at[idx_v.at[2 * j]], buf.at[pl.ds(0, CHUNK)], gsem
        ).start()
        pltpu.make_async_copy(
            table_hbm.at[idx_v.at[2 * j + 1]], buf.at[pl.ds(CHUNK, CHUNK)], gsem
        ).start()

    def g_wait(buf):
        for _ in range(2):
            pltpu.make_async_copy(
                table_hbm.at[idx_v.at[0]], buf.at[pl.ds(0, CHUNK)], gsem
            ).wait()

    def o_start(j, buf):
        pltpu.make_async_copy(
            buf, out_hbm.at[pl.ds(base + j * OCHUNK, OCHUNK)], osem
        ).start()

    def o_wait():
        pltpu.make_async_copy(b0, out_hbm.at[pl.ds(base, OCHUNK)], osem).wait()

    def scale(buf):
        @plsc.parallel_loop(0, OCHUNK, step=1, unroll=4)
        def _row(r):
            for s in range(D_MODEL // 16):
                sl = pl.ds(s * 16, 16)
                buf[r, sl] = buf[r, sl] * SCALE

    g_start(0, b0)
    g_start(1, b1)
    for j in range(NB):
        buf = bufs[j]
        g_wait(buf)
        scale(buf)
        o_start(j, buf)
        if j >= 1:
            o_wait()
        g_start(j + 2, bufs[(j + 2) % NB])

    def group_body(gi, carry):
        j0 = NB * gi
        for b in range(NB):
            j = j0 + b
            buf = bufs[b]
            g_wait(buf)
            scale(buf)
            o_start(j, buf)
            o_wait()
            g_start(j + 2, bufs[(b + 2) % NB])
        return carry

    lax.fori_loop(1, (NOUTER - NB - 1) // NB, group_body, 0)

    for j in range(NOUTER - 4, NOUTER):
        buf = bufs[j % NB]
        g_wait(buf)
        scale(buf)
        o_start(j, buf)
        o_wait()
        if j + 2 < NOUTER:
            g_start(j + 2, bufs[(j + 2) % NB])
    o_wait()


def kernel(indices, W):
    idx = indices.astype(jnp.int32).reshape(NW, NCHUNK, CHUNK)
    out = _embed(W, idx)
    return out.reshape(indices.shape + (D_MODEL,))
